# Initial kernel scaffold; baseline (speedup 1.0000x reference)
import jax
import jax.numpy as jnp
from jax.experimental import pallas as pl


def kernel(X, adj, Wu1, Wv1, b1, Wu2, Wv2, b2):
    raise NotImplementedError("write your pallas kernel here")


# speedup vs baseline: 1.0050x; 1.0050x over previous
import functools

import jax
import jax.numpy as jnp
from jax.experimental import pallas as pl
from jax.experimental.pallas import tpu as pltpu


def _dotT(a, b):
    return jax.lax.dot_general(a, b, (((0,), (0,)), ((), ())),
                               preferred_element_type=jnp.float32)


def _gcn_kernel(x_ref, adj_ref, wu1_ref, wv1_ref, b1_ref, wu2_ref, wv2_ref,
                b2_ref, out_ref, b1s_ref, acc1_ref, a2s_ref, b2s_ref,
                acc2_ref, *, nk, bk):
    p = pl.program_id(0)
    k = pl.program_id(1)

    @pl.when(jnp.logical_and(p == 0, k == 0))
    def _init_phase0():
        b1s_ref[...] = _dotT(x_ref[...], wv1_ref[...])
        acc1_ref[...] = jnp.zeros_like(acc1_ref)

    @pl.when(p == 0)
    def _phase0():
        blk = b1s_ref[pl.ds(k * bk, bk), :]
        acc1_ref[...] += _dotT(blk, adj_ref[...])

    @pl.when(jnp.logical_and(p == 0, k == nk - 1))
    def _end_phase0():
        a1 = _dotT(wu1_ref[...], x_ref[...]) + b1_ref[...]
        h = jnp.maximum(acc1_ref[...] + a1, 0.0)
        a2s_ref[...] = _dotT(wu2_ref[...], h) + b2_ref[...]
        b2s_ref[...] = _dotT(h, wv2_ref[...])
        acc2_ref[...] = jnp.zeros_like(acc2_ref)

    @pl.when(p == 1)
    def _phase1():
        blk = b2s_ref[pl.ds(k * bk, bk), :]
        acc2_ref[...] += _dotT(blk, adj_ref[...])

    @pl.when(jnp.logical_and(p == 1, k == nk - 1))
    def _end_phase1():
        o = jnp.maximum(acc2_ref[...] + a2s_ref[...], 0.0)
        m = jnp.max(o, axis=0, keepdims=True)
        lse = m + jnp.log(jnp.sum(jnp.exp(o - m), axis=0, keepdims=True))
        out_ref[...] = o - lse


def kernel(X, adj, Wu1, Wv1, b1, Wu2, Wv2, b2):
    nfeat, n = X.shape
    nhid = Wu1.shape[1]
    ncls = Wu2.shape[1]
    bk = 200 if n % 200 == 0 else n // 10
    nk = n // bk
    assert bk * nk == n

    grid = (2, nk)
    out = pl.pallas_call(
        functools.partial(_gcn_kernel, nk=nk, bk=bk),
        grid=grid,
        in_specs=[
            pl.BlockSpec((nfeat, n), lambda p, k: (0, 0)),
            pl.BlockSpec((bk, n), lambda p, k: (k, 0)),
            pl.BlockSpec((nfeat, nhid), lambda p, k: (0, 0)),
            pl.BlockSpec((nfeat, nhid), lambda p, k: (0, 0)),
            pl.BlockSpec((nhid, 1), lambda p, k: (0, 0)),
            pl.BlockSpec((nhid, ncls), lambda p, k: (0, 0)),
            pl.BlockSpec((nhid, ncls), lambda p, k: (0, 0)),
            pl.BlockSpec((ncls, 1), lambda p, k: (0, 0)),
        ],
        out_specs=pl.BlockSpec((ncls, n), lambda p, k: (0, 0)),
        out_shape=jax.ShapeDtypeStruct((ncls, n), jnp.float32),
        scratch_shapes=[
            pltpu.VMEM((n, nhid), jnp.float32),
            pltpu.VMEM((nhid, n), jnp.float32),
            pltpu.VMEM((ncls, n), jnp.float32),
            pltpu.VMEM((n, ncls), jnp.float32),
            pltpu.VMEM((ncls, n), jnp.float32),
        ],
    )(X, adj, Wu1, Wv1, b1.reshape(nhid, 1), Wu2, Wv2, b2.reshape(ncls, 1))
    return out.T
